# initial kernel scaffold (unmeasured)
import jax
import jax.numpy as jnp
from jax import lax
from jax.experimental import pallas as pl
from jax.experimental.pallas import tpu as pltpu

N_DEV = 8
TOK = 256
D = 128
H = 256
N_EXP = 16
ROWS = TOK // N_DEV


def kernel(x, router_W, route_idx, expert_W, shared_W):
    def body(x_ref, rw_ref, idx_ref, ew_ref, sw_ref, out_ref,
             partial_ref, comm_ref, send_sems, recv_sems):
        my = lax.axis_index("i")

        xv = x_ref[:, :]
        scores = jnp.dot(xv, rw_ref[:, :], preferred_element_type=jnp.float32)
        s_max = jnp.max(scores, axis=-1, keepdims=True)
        ex = jnp.exp(scores - s_max)
        probs = ex / jnp.sum(ex, axis=-1, keepdims=True)
        e = idx_ref[:, :]
        onehot = lax.broadcasted_iota(jnp.int32, (TOK, N_EXP), 1) == e
        p_sel = jnp.sum(jnp.where(onehot, probs, 0.0), axis=-1, keepdims=True)

        y0 = jnp.dot(xv, ew_ref[0], preferred_element_type=jnp.float32)
        y1 = jnp.dot(xv, ew_ref[1], preferred_element_type=jnp.float32)
        m0 = (e == 2 * my).astype(jnp.float32)
        m1 = (e == 2 * my + 1).astype(jnp.float32)
        partial_ref[:, :] = p_sel * (m0 * y0 + m1 * y1)

        sends = []
        for k in range(1, N_DEV):
            dst = (my + k) % N_DEV
            rdma = pltpu.make_async_remote_copy(
                src_ref=partial_ref.at[pl.ds(dst * ROWS, ROWS), :],
                dst_ref=comm_ref.at[my],
                send_sem=send_sems.at[k],
                recv_sem=recv_sems.at[my],
                device_id=(dst,),
                device_id_type=pl.DeviceIdType.MESH,
            )
            rdma.start()
            sends.append(rdma)

        out_ref[:, :] = (
            jnp.dot(x_ref[pl.ds(my * ROWS, ROWS), :], sw_ref[:, :],
                    preferred_element_type=jnp.float32)
            + partial_ref[pl.ds(my * ROWS, ROWS), :]
        )

        for k in range(1, N_DEV):
            src = (my + k) % N_DEV
            recv = pltpu.make_async_remote_copy(
                src_ref=partial_ref.at[pl.ds(0, ROWS), :],
                dst_ref=comm_ref.at[src],
                send_sem=send_sems.at[k],
                recv_sem=recv_sems.at[src],
                device_id=(src,),
                device_id_type=pl.DeviceIdType.MESH,
            )
            recv.wait_recv()
            out_ref[:, :] += comm_ref[src]

        for rdma in sends:
            rdma.wait_send()

    return pl.pallas_call(
        body,
        out_shape=jax.ShapeDtypeStruct((ROWS, H), jnp.float32),
        in_specs=[pl.BlockSpec(memory_space=pltpu.VMEM)] * 5,
        out_specs=pl.BlockSpec(memory_space=pltpu.VMEM),
        scratch_shapes=[
            pltpu.VMEM((TOK, H), jnp.float32),
            pltpu.VMEM((N_DEV, ROWS, H), jnp.float32),
            pltpu.SemaphoreType.DMA((N_DEV,)),
            pltpu.SemaphoreType.DMA((N_DEV,)),
        ],
        compiler_params=pltpu.CompilerParams(collective_id=0),
    )(x, router_W, route_idx, expert_W, shared_W)


# baseline (device time: 15164 ns/iter reference)
import jax
import jax.numpy as jnp
from jax import lax
from jax.experimental import pallas as pl
from jax.experimental.pallas import tpu as pltpu

N_DEV = 8
TOK = 256
D = 128
H = 256
N_EXP = 16
ROWS = TOK // N_DEV


def kernel(x, router_W, route_idx, expert_W, shared_W):
    def body(x_ref, rw_ref, idx_ref, ew_ref, sw_ref, out_ref,
             partial_ref, comm_ref, send_sems, recv_sems):
        my = lax.axis_index("i")

        xv = x_ref[:, :]
        scores = jnp.dot(xv, rw_ref[:, :], preferred_element_type=jnp.float32)
        s_max = jnp.max(scores, axis=-1, keepdims=True)
        ex = jnp.exp(scores - s_max)
        probs = ex / jnp.sum(ex, axis=-1, keepdims=True)
        e = idx_ref[:, :]
        onehot = lax.broadcasted_iota(jnp.int32, (TOK, N_EXP), 1) == e
        p_sel = jnp.sum(jnp.where(onehot, probs, 0.0), axis=-1, keepdims=True)

        y0 = jnp.dot(xv, ew_ref[0], preferred_element_type=jnp.float32)
        y1 = jnp.dot(xv, ew_ref[1], preferred_element_type=jnp.float32)
        m0 = (e == 2 * my).astype(jnp.float32)
        m1 = (e == 2 * my + 1).astype(jnp.float32)
        partial_ref[:, :] = p_sel * (m0 * y0 + m1 * y1)

        sends = []
        for k in range(1, N_DEV):
            dst = (my + k) % N_DEV
            rdma = pltpu.make_async_remote_copy(
                src_ref=partial_ref.at[pl.ds(dst * ROWS, ROWS), :],
                dst_ref=comm_ref.at[my],
                send_sem=send_sems.at[k],
                recv_sem=recv_sems.at[my],
                device_id=(dst,),
                device_id_type=pl.DeviceIdType.MESH,
            )
            rdma.start()
            sends.append(rdma)

        out_ref[:, :] = (
            jnp.dot(x_ref[pl.ds(my * ROWS, ROWS), :], sw_ref[:, :],
                    preferred_element_type=jnp.float32)
            + partial_ref[pl.ds(my * ROWS, ROWS), :]
        )

        for k in range(1, N_DEV):
            src = (my + k) % N_DEV
            recv = pltpu.make_async_remote_copy(
                src_ref=partial_ref.at[pl.ds(0, ROWS), :],
                dst_ref=comm_ref.at[src],
                send_sem=send_sems.at[k],
                recv_sem=recv_sems.at[src],
                device_id=(src,),
                device_id_type=pl.DeviceIdType.MESH,
            )
            recv.wait_recv()
            out_ref[:, :] += comm_ref[src]

        for rdma in sends:
            rdma.wait_send()

    return pl.pallas_call(
        body,
        out_shape=jax.ShapeDtypeStruct((ROWS, H), jnp.float32),
        in_specs=[pl.BlockSpec(memory_space=pltpu.VMEM)] * 5,
        out_specs=pl.BlockSpec(memory_space=pltpu.VMEM),
        scratch_shapes=[
            pltpu.VMEM((TOK, H), jnp.float32),
            pltpu.VMEM((N_DEV, ROWS, H), jnp.float32),
            pltpu.SemaphoreType.DMA((N_DEV,)),
            pltpu.SemaphoreType.DMA((N_DEV,)),
        ],
    )(x, router_W, route_idx, expert_W, shared_W)


# device time: 10861 ns/iter; 1.3962x vs baseline; 1.3962x over previous
import jax
import jax.numpy as jnp
from jax import lax
from jax.experimental import pallas as pl
from jax.experimental.pallas import tpu as pltpu

N_DEV = 8
TOK = 256
D = 128
H = 256
N_EXP = 16
ROWS = TOK // N_DEV

_SEND_ORDER = (6, 2, 5, 7, 4, 1, 3)


def kernel(x, router_W, route_idx, expert_W, shared_W):
    def body(x_ref, rw_ref, idx_ref, ew_ref, sw_ref, out_ref,
             partial_ref, comm_ref, send_sems, recv_sems):
        my = lax.axis_index("i")

        barrier_sem = pltpu.get_barrier_semaphore()
        for k in range(1, N_DEV):
            pl.semaphore_signal(
                barrier_sem, inc=1,
                device_id=((my + k) % N_DEV,),
                device_id_type=pl.DeviceIdType.MESH,
            )

        xv = x_ref[:, :]
        scores = jnp.dot(xv, rw_ref[:, :], preferred_element_type=jnp.float32)
        s_max = jnp.max(scores, axis=-1, keepdims=True)
        ex = jnp.exp(scores - s_max)
        probs = ex / jnp.sum(ex, axis=-1, keepdims=True)
        e = idx_ref[:, :]
        onehot = lax.broadcasted_iota(jnp.int32, (TOK, N_EXP), 1) == e
        p_sel = jnp.sum(jnp.where(onehot, probs, 0.0), axis=-1, keepdims=True)

        y0 = jnp.dot(xv, ew_ref[0], preferred_element_type=jnp.float32)
        y1 = jnp.dot(xv, ew_ref[1], preferred_element_type=jnp.float32)
        m0 = (e == 2 * my).astype(jnp.float32)
        m1 = (e == 2 * my + 1).astype(jnp.float32)
        partial_ref[:, :] = (p_sel * (m0 * y0 + m1 * y1)).astype(jnp.bfloat16)

        pl.semaphore_wait(barrier_sem, N_DEV - 1)

        sends = []
        for k in _SEND_ORDER:
            dst = (my + k) % N_DEV
            rdma = pltpu.make_async_remote_copy(
                src_ref=partial_ref.at[pl.ds(dst * ROWS, ROWS), :],
                dst_ref=comm_ref.at[N_DEV - k],
                send_sem=send_sems.at[k],
                recv_sem=recv_sems.at[N_DEV - k],
                device_id=(dst,),
                device_id_type=pl.DeviceIdType.MESH,
            )
            rdma.start()
            sends.append(rdma)

        comm_ref[0] = partial_ref[pl.ds(my * ROWS, ROWS), :]
        shared = jnp.dot(x_ref[pl.ds(my * ROWS, ROWS), :], sw_ref[:, :],
                         preferred_element_type=jnp.float32)

        for s in range(1, N_DEV):
            recv = pltpu.make_async_remote_copy(
                src_ref=partial_ref.at[pl.ds(0, ROWS), :],
                dst_ref=comm_ref.at[s],
                send_sem=send_sems.at[s],
                recv_sem=recv_sems.at[s],
                device_id=((my + s) % N_DEV,),
                device_id_type=pl.DeviceIdType.MESH,
            )
            recv.wait_recv()

        out_ref[:, :] = shared + jnp.sum(
            comm_ref[:, :, :].astype(jnp.float32), axis=0)

        for rdma in sends:
            rdma.wait_send()

    return pl.pallas_call(
        body,
        out_shape=jax.ShapeDtypeStruct((ROWS, H), jnp.float32),
        in_specs=[pl.BlockSpec(memory_space=pltpu.VMEM)] * 5,
        out_specs=pl.BlockSpec(memory_space=pltpu.VMEM),
        scratch_shapes=[
            pltpu.VMEM((TOK, H), jnp.bfloat16),
            pltpu.VMEM((N_DEV, ROWS, H), jnp.bfloat16),
            pltpu.SemaphoreType.DMA((N_DEV,)),
            pltpu.SemaphoreType.DMA((N_DEV,)),
        ],
        compiler_params=pltpu.CompilerParams(collective_id=0),
    )(x, router_W, route_idx, expert_W, shared_W)
